# baseline (speedup 1.0000x reference)
import functools

import jax
import jax.numpy as jnp
from jax import lax
from jax.experimental import pallas as pl
from jax.experimental.pallas import tpu as pltpu
from jax.experimental.pallas import tpu_sc as plsc

_VOCAB = 100000
_HIDDEN = 768
_BATCH = 4
_SEQ = 2048
_B = _BATCH * _SEQ

_NC = 2
_NS = 16
_NW = _NC * _NS
_S_PER_W = _SEQ // _NW
_ROWS_W = _S_PER_W * _BATCH
_CHUNK = 32
_L = 16
_NG = _HIDDEN // _L


def _emb_body(src_hbm, seg_hbm, tok_tab_hbm, seg_tab_hbm, pos_hbm, out_hbm,
              idx_all, seg_all, seg_tab_v, pos_v, buf0, buf1,
              semg0, semg1, semw0, semw1):
    buf = [buf0, buf1]
    semg = [semg0, semg1]
    semw = [semw0, semw1]
    wid = lax.axis_index("s") * _NC + lax.axis_index("c")
    s_base = pl.multiple_of(wid * _S_PER_W, _S_PER_W)
    with jax.named_scope("stage"):
        for b in range(_BATCH):
            fo = pl.multiple_of(b * _SEQ + s_base, _S_PER_W)
            pltpu.sync_copy(src_hbm.at[pl.ds(fo, _S_PER_W)],
                            idx_all.at[pl.ds(b * _S_PER_W, _S_PER_W)])
            pltpu.sync_copy(seg_hbm.at[pl.ds(fo, _S_PER_W)],
                            seg_all.at[pl.ds(b * _S_PER_W, _S_PER_W)])
        pltpu.sync_copy(seg_tab_hbm, seg_tab_v)
        pltpu.sync_copy(pos_hbm.at[pl.ds(s_base, _S_PER_W)], pos_v)

    fold_scope = jax.named_scope("fold")
    fold_scope.__enter__()

    def fold_body(g, carry):
        goff = pl.multiple_of(g * _L, _L)
        s0 = seg_tab_v[0, pl.ds(goff, _L)]

        def r_body(r, inner):
            plsc.addupdate(pos_v.at[r, pl.ds(goff, _L)], s0)
            return inner

        lax.fori_loop(0, _S_PER_W, r_body, 0)
        return carry

    lax.fori_loop(0, _NG, fold_body, 0)
    fold_scope.__exit__(None, None, None)

    nt = _S_PER_W // _CHUNK
    n = _BATCH * nt

    def hbm_off(c):
        b, t = divmod(c, nt)
        return pl.multiple_of(b * _SEQ + s_base + t * _CHUNK, _CHUNK)

    def start_gather(c):
        return pltpu.async_copy(
            tok_tab_hbm.at[idx_all.at[pl.ds(c * _CHUNK, _CHUNK)]],
            buf[c % 2], semg[c % 2])

    def add_posseg(c):
        buf_ref = buf[c % 2]
        pos_base = (c % nt) * _CHUNK
        soff = c * _CHUNK

        def g_body(g, carry):
            goff = pl.multiple_of(g * _L, _L)
            s1 = seg_tab_v[1, pl.ds(goff, _L)]
            s0 = seg_tab_v[0, pl.ds(goff, _L)]
            d = s1 - s0

            def blk_body(rb, inner):
                segf16 = seg_all[pl.ds(soff + rb * _L, _L)].astype(
                    jnp.float32)
                for j in range(_L):
                    r = rb * _L + j
                    segf = jnp.full((_L,), segf16[j])
                    p = pos_v[pos_base + r, pl.ds(goff, _L)]
                    plsc.addupdate(buf_ref.at[r, pl.ds(goff, _L)],
                                   p + segf * d)
                return inner

            lax.fori_loop(0, _CHUNK // _L, blk_body, 0)
            return carry

        lax.fori_loop(0, _NG, g_body, 0)

    gcp = [None, None]
    wcp = [None, None]
    gcp[0] = start_gather(0)
    for c in range(n):
        cur = c % 2
        nxt = (c + 1) % 2
        if c + 1 < n:
            if wcp[nxt] is not None:
                with jax.named_scope(f"wwait{c}"):
                    wcp[nxt].wait()
                wcp[nxt] = None
            gcp[nxt] = start_gather(c + 1)
        with jax.named_scope(f"gwait{c}"):
            gcp[cur].wait()
        with jax.named_scope(f"compute{c}"):
            add_posseg(c)
        wcp[cur] = pltpu.async_copy(
            buf[cur], out_hbm.at[pl.ds(hbm_off(c), _CHUNK)], semw[cur])
    for w in wcp:
        if w is not None:
            w.wait()


@jax.jit
def _emb(src_flat, seg_flat, token_table, segment_table, pos_flat):
    mesh = plsc.VectorSubcoreMesh(
        core_axis_name="c", subcore_axis_name="s",
        num_cores=_NC, num_subcores=_NS)
    run = functools.partial(
        pl.kernel,
        out_type=jax.ShapeDtypeStruct((_B, _HIDDEN), jnp.float32),
        mesh=mesh,
        scratch_types=[
            pltpu.VMEM((_ROWS_W,), jnp.int32),
            pltpu.VMEM((_ROWS_W,), jnp.int32),
            pltpu.VMEM((2, _HIDDEN), jnp.float32),
            pltpu.VMEM((_S_PER_W, _HIDDEN), jnp.float32),
            pltpu.VMEM((_CHUNK, _HIDDEN), jnp.float32),
            pltpu.VMEM((_CHUNK, _HIDDEN), jnp.float32),
            pltpu.SemaphoreType.DMA,
            pltpu.SemaphoreType.DMA,
            pltpu.SemaphoreType.DMA,
            pltpu.SemaphoreType.DMA,
        ],
    )(_emb_body)
    return run(src_flat, seg_flat, token_table, segment_table, pos_flat)


def kernel(source, segment, token_table, segment_table, pos_emb):
    src_flat = source.reshape(_B).astype(jnp.int32)
    seg_flat = segment.reshape(_B).astype(jnp.int32)
    pos_flat = pos_emb.reshape(pos_emb.shape[1], _HIDDEN)
    out = _emb(src_flat, seg_flat, token_table, segment_table, pos_flat)
    return out.reshape(_BATCH, _SEQ, _HIDDEN)


# speedup vs baseline: 1.7162x; 1.7162x over previous
import functools

import jax
import jax.numpy as jnp
from jax import lax
from jax.experimental import pallas as pl
from jax.experimental.pallas import tpu as pltpu
from jax.experimental.pallas import tpu_sc as plsc

_VOCAB = 100000
_HIDDEN = 768
_BATCH = 4
_SEQ = 2048
_B = _BATCH * _SEQ

_NC = 2
_NS = 16
_NW = _NC * _NS
_S_PER_W = _SEQ // _NW
_ROWS_W = _S_PER_W * _BATCH
_CHUNK = 32
_L = 16
_NG = _HIDDEN // _L


def _emb_body(src_hbm, seg_hbm, tok_tab_hbm, seg_tab_hbm, pos_hbm, out_hbm,
              idx_all, seg_all, seg_tab_v, pos_v, buf0, buf1,
              semg0, semg1, semw0, semw1):
    buf = [buf0, buf1]
    semg = [semg0, semg1]
    semw = [semw0, semw1]
    wid = lax.axis_index("s") * _NC + lax.axis_index("c")
    s_base = pl.multiple_of(wid * _S_PER_W, _S_PER_W)
    with jax.named_scope("stage"):
        stage_cps = []
        for b in range(_BATCH):
            fo = pl.multiple_of(b * _SEQ + s_base, _S_PER_W)
            stage_cps.append(pltpu.async_copy(
                src_hbm.at[pl.ds(fo, _S_PER_W)],
                idx_all.at[pl.ds(b * _S_PER_W, _S_PER_W)], semw0))
            stage_cps.append(pltpu.async_copy(
                seg_hbm.at[pl.ds(fo, _S_PER_W)],
                seg_all.at[pl.ds(b * _S_PER_W, _S_PER_W)], semw0))
        stage_cps.append(pltpu.async_copy(seg_tab_hbm, seg_tab_v, semw1))
        stage_cps.append(pltpu.async_copy(
            pos_hbm.at[pl.ds(s_base, _S_PER_W)], pos_v, semw1))
        for cp in stage_cps:
            cp.wait()

    nt = _S_PER_W // _CHUNK
    n = _BATCH * nt

    def hbm_off(c):
        b, t = divmod(c, nt)
        return pl.multiple_of(b * _SEQ + s_base + t * _CHUNK, _CHUNK)

    def start_gather(c):
        return pltpu.async_copy(
            tok_tab_hbm.at[idx_all.at[pl.ds(c * _CHUNK, _CHUNK)]],
            buf[c % 2], semg[c % 2])

    def add_posseg(c):
        buf_ref = buf[c % 2]
        pos_ref = pos_v.at[pl.ds((c % nt) * _CHUNK, _CHUNK)]
        soff = c * _CHUNK

        def g_body(g, carry):
            goff = pl.multiple_of(g * _L, _L)
            s1 = seg_tab_v[1, pl.ds(goff, _L)]
            s0 = seg_tab_v[0, pl.ds(goff, _L)]
            d = s1 - s0

            for rb in range(_CHUNK // _L):
                segf16 = seg_all[pl.ds(soff + rb * _L, _L)].astype(
                    jnp.float32)
                for j in range(_L):
                    r = rb * _L + j
                    segf = jnp.full((_L,), segf16[j])
                    p = pos_ref[r, pl.ds(goff, _L)]
                    plsc.addupdate(buf_ref.at[r, pl.ds(goff, _L)],
                                   p + s0 + segf * d)
            return carry

        lax.fori_loop(0, _NG, g_body, 0)

    gcp = [None, None]
    wcp = [None, None]
    gcp[0] = start_gather(0)
    for c in range(n):
        cur = c % 2
        nxt = (c + 1) % 2
        if c + 1 < n:
            if wcp[nxt] is not None:
                with jax.named_scope(f"wwait{c}"):
                    wcp[nxt].wait()
                wcp[nxt] = None
            gcp[nxt] = start_gather(c + 1)
        with jax.named_scope(f"gwait{c}"):
            gcp[cur].wait()
        with jax.named_scope(f"compute{c}"):
            add_posseg(c)
        wcp[cur] = pltpu.async_copy(
            buf[cur], out_hbm.at[pl.ds(hbm_off(c), _CHUNK)], semw[cur])
    for w in wcp:
        if w is not None:
            w.wait()


@jax.jit
def _emb(src_flat, seg_flat, token_table, segment_table, pos_flat):
    mesh = plsc.VectorSubcoreMesh(
        core_axis_name="c", subcore_axis_name="s",
        num_cores=_NC, num_subcores=_NS)
    run = functools.partial(
        pl.kernel,
        out_type=jax.ShapeDtypeStruct((_B, _HIDDEN), jnp.float32),
        mesh=mesh,
        scratch_types=[
            pltpu.VMEM((_ROWS_W,), jnp.int32),
            pltpu.VMEM((_ROWS_W,), jnp.int32),
            pltpu.VMEM((2, _HIDDEN), jnp.float32),
            pltpu.VMEM((_S_PER_W, _HIDDEN), jnp.float32),
            pltpu.VMEM((_CHUNK, _HIDDEN), jnp.float32),
            pltpu.VMEM((_CHUNK, _HIDDEN), jnp.float32),
            pltpu.SemaphoreType.DMA,
            pltpu.SemaphoreType.DMA,
            pltpu.SemaphoreType.DMA,
            pltpu.SemaphoreType.DMA,
        ],
    )(_emb_body)
    return run(src_flat, seg_flat, token_table, segment_table, pos_flat)


def kernel(source, segment, token_table, segment_table, pos_emb):
    src_flat = source.reshape(_B).astype(jnp.int32)
    seg_flat = segment.reshape(_B).astype(jnp.int32)
    pos_flat = pos_emb.reshape(pos_emb.shape[1], _HIDDEN)
    out = _emb(src_flat, seg_flat, token_table, segment_table, pos_flat)
    return out.reshape(_BATCH, _SEQ, _HIDDEN)


# speedup vs baseline: 1.9136x; 1.1150x over previous
import functools

import jax
import jax.numpy as jnp
from jax import lax
from jax.experimental import pallas as pl
from jax.experimental.pallas import tpu as pltpu
from jax.experimental.pallas import tpu_sc as plsc

_VOCAB = 100000
_HIDDEN = 768
_BATCH = 4
_SEQ = 2048
_B = _BATCH * _SEQ

_NC = 2
_NS = 16
_NW = _NC * _NS
_S_PER_W = _SEQ // _NW
_ROWS_W = _S_PER_W * _BATCH
_CHUNK = 32
_NBUF = 3
_L = 16
_NG = _HIDDEN // _L


def _emb_body(src_hbm, seg_hbm, tok_tab_hbm, seg_tab_hbm, pos_hbm, out_hbm,
              idx_all, seg_all, seg_tab_v, pos_v, buf0, buf1, buf2,
              semg0, semg1, semg2, semw0, semw1, semw2, semst):
    buf = [buf0, buf1, buf2]
    semg = [semg0, semg1, semg2]
    semw = [semw0, semw1, semw2]
    wid = lax.axis_index("s") * _NC + lax.axis_index("c")
    s_base = pl.multiple_of(wid * _S_PER_W, _S_PER_W)

    nt = _S_PER_W // _CHUNK
    n = _BATCH * nt

    def hbm_off(c):
        b, t = divmod(c, nt)
        return pl.multiple_of(b * _SEQ + s_base + t * _CHUNK, _CHUNK)

    def start_gather(c):
        return pltpu.async_copy(
            tok_tab_hbm.at[idx_all.at[pl.ds(c * _CHUNK, _CHUNK)]],
            buf[c % _NBUF], semg[c % _NBUF])

    icps = []
    rest = []
    for b in range(_BATCH):
        fo = pl.multiple_of(b * _SEQ + s_base, _S_PER_W)
        icps.append(pltpu.async_copy(
            src_hbm.at[pl.ds(fo, _S_PER_W)],
            idx_all.at[pl.ds(b * _S_PER_W, _S_PER_W)], semst))
        rest.append(pltpu.async_copy(
            seg_hbm.at[pl.ds(fo, _S_PER_W)],
            seg_all.at[pl.ds(b * _S_PER_W, _S_PER_W)], semw0))
    rest.append(pltpu.async_copy(seg_tab_hbm, seg_tab_v, semw1))
    rest.append(pltpu.async_copy(
        pos_hbm.at[pl.ds(s_base, _S_PER_W)], pos_v, semw2))

    gcp = [None] * _NBUF
    wcp = [None] * _NBUF
    icps[0].wait()
    gcp[0] = start_gather(0)
    gcp[1] = start_gather(1)
    for cp in icps[1:]:
        cp.wait()
    for cp in rest:
        cp.wait()

    def add_posseg(c):
        buf_ref = buf[c % _NBUF]
        pos_ref = pos_v.at[pl.ds((c % nt) * _CHUNK, _CHUNK)]
        soff = c * _CHUNK

        def g_body(g, carry):
            goff = pl.multiple_of(g * _L, _L)
            s1 = seg_tab_v[1, pl.ds(goff, _L)]
            s0 = seg_tab_v[0, pl.ds(goff, _L)]
            d = s1 - s0

            for rb in range(_CHUNK // _L):
                segf16 = seg_all[pl.ds(soff + rb * _L, _L)].astype(
                    jnp.float32)
                for j in range(_L):
                    r = rb * _L + j
                    segf = jnp.full((_L,), segf16[j])
                    p = pos_ref[r, pl.ds(goff, _L)]
                    plsc.addupdate(buf_ref.at[r, pl.ds(goff, _L)],
                                   p + s0 + segf * d)
            return carry

        lax.fori_loop(0, _NG, g_body, 0)

    for c in range(n):
        cur = c % _NBUF
        gcp[cur].wait()
        add_posseg(c)
        wcp[cur] = pltpu.async_copy(
            buf[cur], out_hbm.at[pl.ds(hbm_off(c), _CHUNK)], semw[cur])
        if c + 2 < n:
            k = (c + 2) % _NBUF
            if wcp[k] is not None:
                wcp[k].wait()
                wcp[k] = None
            gcp[k] = start_gather(c + 2)
    for w in wcp:
        if w is not None:
            w.wait()


@jax.jit
def _emb(src_flat, seg_flat, token_table, segment_table, pos_flat):
    mesh = plsc.VectorSubcoreMesh(
        core_axis_name="c", subcore_axis_name="s",
        num_cores=_NC, num_subcores=_NS)
    run = functools.partial(
        pl.kernel,
        out_type=jax.ShapeDtypeStruct((_B, _HIDDEN), jnp.float32),
        mesh=mesh,
        scratch_types=[
            pltpu.VMEM((_ROWS_W,), jnp.int32),
            pltpu.VMEM((_ROWS_W,), jnp.int32),
            pltpu.VMEM((2, _HIDDEN), jnp.float32),
            pltpu.VMEM((_S_PER_W, _HIDDEN), jnp.float32),
            pltpu.VMEM((_CHUNK, _HIDDEN), jnp.float32),
            pltpu.VMEM((_CHUNK, _HIDDEN), jnp.float32),
            pltpu.VMEM((_CHUNK, _HIDDEN), jnp.float32),
            pltpu.SemaphoreType.DMA,
            pltpu.SemaphoreType.DMA,
            pltpu.SemaphoreType.DMA,
            pltpu.SemaphoreType.DMA,
            pltpu.SemaphoreType.DMA,
            pltpu.SemaphoreType.DMA,
            pltpu.SemaphoreType.DMA,
        ],
    )(_emb_body)
    return run(src_flat, seg_flat, token_table, segment_table, pos_flat)


def kernel(source, segment, token_table, segment_table, pos_emb):
    src_flat = source.reshape(_B).astype(jnp.int32)
    seg_flat = segment.reshape(_B).astype(jnp.int32)
    pos_flat = pos_emb.reshape(pos_emb.shape[1], _HIDDEN)
    out = _emb(src_flat, seg_flat, token_table, segment_table, pos_flat)
    return out.reshape(_BATCH, _SEQ, _HIDDEN)
